# baseline (speedup 1.0000x reference)
import jax
import jax.numpy as jnp
from jax.experimental import pallas as pl
from jax.experimental.pallas import tpu as pltpu

BATCH = 64
BLOCK = 72
MAX_PATH = 5
NUM_NODES = BATCH * BLOCK


def _spatial_kernel(b_ref, paths_ref, out_ref):
    i = pl.program_id(0)
    p = paths_ref[0]
    lengths = jnp.sum((p != -1).astype(jnp.int32), axis=0)
    vals = jnp.zeros((BLOCK, BLOCK), dtype=jnp.float32)
    for k in range(MAX_PATH + 1):
        vals = jnp.where(lengths == k, b_ref[k], vals)
    out_ref[...] = jnp.zeros((BLOCK, NUM_NODES), dtype=jnp.float32)
    start = i * BLOCK
    atile = jnp.minimum(start // 128, (NUM_NODES - 256) // 128)
    astart = atile * 128
    off = start - astart
    tiled4 = jnp.concatenate([vals] * 4, axis=1)
    rolled = pltpu.roll(tiled4, off % BLOCK, axis=1)
    window = rolled[:, :256]
    c = jax.lax.broadcasted_iota(jnp.int32, (BLOCK, 256), 1)
    mask = (c >= off) & (c < off + BLOCK)
    out_ref[:, pl.ds(astart, 256)] = jnp.where(mask, window, 0.0)


def kernel(x, paths, b):
    del x
    p32 = jnp.transpose(paths.astype(jnp.int32), (0, 3, 1, 2))
    return pl.pallas_call(
        _spatial_kernel,
        grid=(BATCH,),
        in_specs=[
            pl.BlockSpec(memory_space=pltpu.SMEM),
            pl.BlockSpec((1, MAX_PATH, BLOCK, BLOCK), lambda i: (i, 0, 0, 0)),
        ],
        out_specs=pl.BlockSpec((BLOCK, NUM_NODES), lambda i: (i, 0)),
        out_shape=jax.ShapeDtypeStruct((NUM_NODES, NUM_NODES), jnp.float32),
        compiler_params=pltpu.CompilerParams(
            dimension_semantics=("parallel",),
        ),
    )(b, p32)


# speedup vs baseline: 1.6937x; 1.6937x over previous
import jax
import jax.numpy as jnp
from jax.experimental import pallas as pl
from jax.experimental.pallas import tpu as pltpu

BATCH = 64
BLOCK = 72
MAX_PATH = 5
NUM_NODES = BATCH * BLOCK
BLOCKS_PER = 4
ROWS_PER = BLOCK * BLOCKS_PER
GRID = BATCH // BLOCKS_PER


def _spatial_kernel(b_ref, paths_ref, out_ref):
    g = pl.program_id(0)
    out_ref[...] = jnp.zeros((ROWS_PER, NUM_NODES), dtype=jnp.float32)
    for r in range(BLOCKS_PER):
        i = g * BLOCKS_PER + r
        p = paths_ref[r]
        lengths = jnp.sum((p != -1).astype(jnp.int32), axis=0)
        vals = jnp.zeros((BLOCK, BLOCK), dtype=jnp.float32)
        for k in range(MAX_PATH + 1):
            vals = jnp.where(lengths == k, b_ref[k], vals)
        start = i * BLOCK
        atile = jnp.minimum(start // 128, (NUM_NODES - 256) // 128)
        astart = atile * 128
        off = start - astart
        tiled4 = jnp.concatenate([vals] * 4, axis=1)
        rolled = pltpu.roll(tiled4, off % BLOCK, axis=1)
        window = rolled[:, :256]
        c = jax.lax.broadcasted_iota(jnp.int32, (BLOCK, 256), 1)
        mask = (c >= off) & (c < off + BLOCK)
        out_ref[r * BLOCK:(r + 1) * BLOCK, pl.ds(astart, 256)] = (
            jnp.where(mask, window, 0.0))


def kernel(x, paths, b):
    del x
    p32 = jnp.transpose(paths.astype(jnp.int32), (0, 3, 1, 2))
    return pl.pallas_call(
        _spatial_kernel,
        grid=(GRID,),
        in_specs=[
            pl.BlockSpec(memory_space=pltpu.SMEM),
            pl.BlockSpec((BLOCKS_PER, MAX_PATH, BLOCK, BLOCK),
                         lambda i: (i, 0, 0, 0)),
        ],
        out_specs=pl.BlockSpec((ROWS_PER, NUM_NODES), lambda i: (i, 0)),
        out_shape=jax.ShapeDtypeStruct((NUM_NODES, NUM_NODES), jnp.float32),
        compiler_params=pltpu.CompilerParams(
            dimension_semantics=("parallel",),
        ),
    )(b, p32)


# speedup vs baseline: 1.8326x; 1.0820x over previous
import jax
import jax.numpy as jnp
from jax.experimental import pallas as pl
from jax.experimental.pallas import tpu as pltpu

BATCH = 64
BLOCK = 72
MAX_PATH = 5
NUM_NODES = BATCH * BLOCK
BLOCKS_PER = 8
ROWS_PER = BLOCK * BLOCKS_PER
GRID = BATCH // BLOCKS_PER


def _spatial_kernel(b_ref, paths_ref, out_ref):
    g = pl.program_id(0)
    out_ref[...] = jnp.zeros((ROWS_PER, NUM_NODES), dtype=jnp.float32)
    for r in range(BLOCKS_PER):
        i = g * BLOCKS_PER + r
        p = paths_ref[r]
        lengths = jnp.sum((p != -1).astype(jnp.int32), axis=0)
        vals = jnp.zeros((BLOCK, BLOCK), dtype=jnp.float32)
        for k in range(MAX_PATH + 1):
            vals = jnp.where(lengths == k, b_ref[k], vals)
        start = i * BLOCK
        atile = jnp.minimum(start // 128, (NUM_NODES - 256) // 128)
        astart = atile * 128
        off = start - astart
        tiled4 = jnp.concatenate([vals] * 4, axis=1)
        rolled = pltpu.roll(tiled4, off % BLOCK, axis=1)
        window = rolled[:, :256]
        c = jax.lax.broadcasted_iota(jnp.int32, (BLOCK, 256), 1)
        mask = (c >= off) & (c < off + BLOCK)
        out_ref[r * BLOCK:(r + 1) * BLOCK, pl.ds(astart, 256)] = (
            jnp.where(mask, window, 0.0))


def kernel(x, paths, b):
    del x
    p32 = jnp.transpose(paths.astype(jnp.int32), (0, 3, 1, 2))
    return pl.pallas_call(
        _spatial_kernel,
        grid=(GRID,),
        in_specs=[
            pl.BlockSpec(memory_space=pltpu.SMEM),
            pl.BlockSpec((BLOCKS_PER, MAX_PATH, BLOCK, BLOCK),
                         lambda i: (i, 0, 0, 0)),
        ],
        out_specs=pl.BlockSpec((ROWS_PER, NUM_NODES), lambda i: (i, 0)),
        out_shape=jax.ShapeDtypeStruct((NUM_NODES, NUM_NODES), jnp.float32),
        compiler_params=pltpu.CompilerParams(
            dimension_semantics=("parallel",),
        ),
    )(b, p32)
